# baseline (speedup 1.0000x reference)
import functools

import jax
import jax.numpy as jnp
from jax import lax
from jax.experimental import pallas as pl
from jax.experimental.pallas import tpu as pltpu
from jax.experimental.pallas import tpu_sc as plsc

_NC = 2
_NS = 16
_L = 16
_NW = _NC * _NS
_CH = 128
_CHA = 64
_NBUF = 4
_DIM = 128
_EPS = 1e-06


def _mesh():
    return plsc.VectorSubcoreMesh(core_axis_name="c", subcore_axis_name="s")


def _sc_params(tc_tiling=True):
    return pltpu.CompilerParams(needs_layout_passes=False,
                                use_tc_tiling_on_sc=tc_tiling)


def _make_deg_kernel(n_chunks, n_pad):
    trows = n_pad // _NS

    @functools.partial(
        pl.kernel,
        out_type=jax.ShapeDtypeStruct((_NC, n_pad), jnp.float32),
        mesh=_mesh(),
        compiler_params=_sc_params(),
        scratch_types=[
            pltpu.VMEM((n_chunks, _CH), jnp.int32),
            pltpu.VMEM((n_chunks, _CH), jnp.float32),
            pltpu.VMEM((trows,), jnp.float32),
            pltpu.VMEM_SHARED((n_pad,), jnp.float32),
            pltpu.SemaphoreType.DMA,
        ],
    )
    def deg_kernel(dst_hbm, ew_hbm, out_hbm, dst_v, ew_v, zbuf, acc_sh, sem):
        cid = lax.axis_index("c")
        sid = lax.axis_index("s")
        wid = sid * _NC + cid

        def zinit(i, carry):
            zbuf[pl.ds(i * _L, _L)] = jnp.zeros((_L,), jnp.float32)
            return carry

        lax.fori_loop(0, trows // _L, zinit, 0)
        pltpu.sync_copy(zbuf, acc_sh.at[pl.ds(sid * trows, trows)])
        pltpu.sync_copy(dst_hbm.at[wid], dst_v)
        pltpu.sync_copy(ew_hbm.at[wid], ew_v)
        plsc.subcore_barrier()

        def fire(c, carry):
            pltpu.async_copy(ew_v.at[c], acc_sh.at[dst_v.at[c]], sem,
                             add=True)
            return carry

        lax.fori_loop(0, n_chunks, fire, 0)

        def drain(c, carry):
            pltpu.make_async_copy(
                ew_v.at[c], acc_sh.at[dst_v.at[c]], sem).wait()
            return carry

        lax.fori_loop(0, n_chunks, drain, 0)
        plsc.subcore_barrier()
        pltpu.sync_copy(acc_sh.at[pl.ds(sid * trows, trows)],
                        out_hbm.at[cid, pl.ds(sid * trows, trows)])

    return deg_kernel


def _make_agg_kernel(k0, k1, n_pad):
    trows = n_pad // _NS
    zfull = trows // _CHA
    zrem = trows - zfull * _CHA

    @functools.partial(
        pl.kernel,
        out_type=jax.ShapeDtypeStruct((_NC, n_pad, _DIM), jnp.float32),
        mesh=_mesh(),
        compiler_params=_sc_params(tc_tiling=False),
        scratch_types=[pltpu.VMEM((_CHA, _DIM), jnp.bfloat16)
                       for _ in range(_NBUF)]
        + [pltpu.VMEM((_CHA, _DIM), jnp.float32) for _ in range(_NBUF)]
        + [pltpu.VMEM((3, _CHA), jnp.int32) for _ in range(_NBUF)]
        + [pltpu.VMEM_SHARED((n_pad, _DIM), jnp.float32)]
        + [pltpu.SemaphoreType.DMA for _ in range(3 * _NBUF)],
    )
    def agg_kernel(pk_hbm, yw_hbm, out_hbm, *rest):
        rbf = rest[:_NBUF]
        rows = rest[_NBUF:2 * _NBUF]
        pk = rest[2 * _NBUF:3 * _NBUF]
        acc_sh = rest[3 * _NBUF]
        sem_g = rest[3 * _NBUF + 1:4 * _NBUF + 1]
        sem_s = rest[4 * _NBUF + 1:5 * _NBUF + 1]
        sem_i = rest[5 * _NBUF + 1:]
        cid = lax.axis_index("c")
        sid = lax.axis_index("s")
        nck = jnp.where(cid == 0, k0, k1)
        base = jnp.where(cid == 0, sid * k0, _NS * k0 + sid * k1)
        n_outer = nck // _NBUF

        def zinit(r, carry):
            for k in range(_DIM // _L):
                rows[0][r, pl.ds(k * _L, _L)] = jnp.zeros((_L,), jnp.float32)
            return carry

        lax.fori_loop(0, _CHA, zinit, 0)
        for t in range(zfull):
            pltpu.sync_copy(rows[0],
                            acc_sh.at[pl.ds(sid * trows + t * _CHA, _CHA)])
        if zrem:
            pltpu.sync_copy(
                rows[0].at[pl.ds(0, zrem)],
                acc_sh.at[pl.ds(sid * trows + zfull * _CHA, zrem)])
        plsc.subcore_barrier()

        def pk_load(c, b):
            pltpu.async_copy(pk_hbm.at[base + c], pk[b], sem_i[b])

        def wait_pk(c, b):
            pltpu.make_async_copy(pk_hbm.at[base + c], pk[b],
                                  sem_i[b]).wait()

        def gather(c, b):
            pltpu.async_copy(yw_hbm.at[pk[b].at[0]], rbf[b], sem_g[b])

        def wait_gather(c, b):
            pltpu.make_async_copy(
                yw_hbm.at[pk[b].at[0]], rbf[b], sem_g[b]).wait()

        def scatter(c, b):
            pltpu.async_copy(rows[b], acc_sh.at[pk[b].at[1]], sem_s[b],
                             add=True)

        def wait_scatter(c, b):
            pltpu.make_async_copy(
                rows[b], acc_sh.at[pk[b].at[1]], sem_s[b]).wait()

        pk_load(0, 0)
        pk_load(1, 1)
        wait_pk(0, 0)
        gather(0, 0)

        def outer(g, carry):
            for b in range(_NBUF):
                c = g * _NBUF + b
                wait_gather(c, b)

                @pl.when(c >= 2)
                def _():
                    wait_scatter(c - 2, (b + 2) % _NBUF)

                @pl.when(c + 2 < nck)
                def _():
                    pk_load(c + 2, (b + 2) % _NBUF)

                @pl.when(c + 1 < nck)
                def _():
                    wait_pk(c + 1, (b + 1) % _NBUF)
                    gather(c + 1, (b + 1) % _NBUF)

                @plsc.parallel_loop(0, _CHA // _L, unroll=2)
                def grp(gr):
                    for j in range(_L):
                        r = gr * _L + j
                        sp = plsc.bitcast(
                            plsc.load_gather(
                                pk[b].at[2],
                                [jnp.full((_L,), r, jnp.int32)]),
                            jnp.float32)
                        for k in range(_DIM // (2 * _L)):
                            pair = plsc.bitcast(
                                rbf[b][r, pl.ds(k * 2 * _L, 2 * _L)],
                                jnp.int32)
                            lo = plsc.bitcast(
                                lax.shift_left(pair, 16), jnp.float32)
                            hi = plsc.bitcast(
                                lax.bitwise_and(
                                    pair, jnp.int32(-65536)), jnp.float32)
                            rows[b][r, pl.ds(k * 2 * _L, _L)] = lo * sp
                            rows[b][r, pl.ds(k * 2 * _L + _L, _L)] = hi * sp

                scatter(c, b)
            return carry

        lax.fori_loop(0, n_outer, outer, 0)
        wait_scatter(nck - 2, (_NBUF - 2) % _NBUF)
        wait_scatter(nck - 1, _NBUF - 1)
        plsc.subcore_barrier()
        pltpu.sync_copy(acc_sh.at[pl.ds(sid * trows, trows)],
                        out_hbm.at[cid, pl.ds(sid * trows, trows)])

    return agg_kernel


def _dis_from_parts(degp_block):
    deg = 1.0 + degp_block[0, :] + degp_block[1, :]
    safe = jnp.where(deg > 0, deg, 1.0)
    return jnp.where(deg > 0, lax.rsqrt(safe), 0.0)


def _k1a_body(x_ref, w_ref, xw_ref):
    xw_ref[...] = lax.dot_general(x_ref[...], w_ref[...],
                                  (((1,), (1,)), ((), ())),
                                  preferred_element_type=jnp.float32)


def _k1b_body(xw_ref, degp_ref, yw_ref, ywb_ref):
    dis = _dis_from_parts(degp_ref[...])
    yw = xw_ref[...] * dis[:, None]
    yw_ref[...] = yw
    yb = yw.astype(jnp.bfloat16)
    blk = yb.shape[0]
    ywb_ref[...] = yb.reshape(
        blk, _DIM // 32, 2, 16).swapaxes(2, 3).reshape(blk, _DIM)


def _k3_body(acc_ref, yw_ref, degp_ref, b_ref, o_ref):
    dis = _dis_from_parts(degp_ref[...])
    out = (acc_ref[0] + acc_ref[1] + yw_ref[...]) * dis[:, None] + b_ref[...]
    mean = jnp.mean(out, axis=1, keepdims=True)
    cen = out - mean
    var = jnp.mean(cen * cen, axis=1, keepdims=True)
    xn = cen * lax.rsqrt(var + _EPS)
    o_ref[...] = jnp.where(xn >= 0, xn, 0.01 * xn)


def kernel(x_src, x_tar, edge_index, edge_weight, W, b):
    n_src = x_src.shape[0]
    n_tar = x_tar.shape[0]
    n = n_src + n_tar
    e = edge_index.shape[1]

    gran = _NW * _CHA * _NBUF
    e_pad = ((e + gran - 1) // gran) * gran
    nd_chunks = e_pad // (_NW * _CH)
    npd = ((n + 255) // 256) * 256
    npa = ((n + 127) // 128) * 128

    src = edge_index[0]
    dst = edge_index[1]
    ep = e_pad - e
    srcp = jnp.concatenate([src, jnp.zeros((ep,), src.dtype)])
    dstp = jnp.concatenate([dst, jnp.zeros((ep,), dst.dtype)])
    ewp = jnp.concatenate(
        [edge_weight, jnp.zeros((ep,), edge_weight.dtype)])
    tot_chunks = e_pad // _CHA
    per_pair = tot_chunks // _NS
    k1 = max(_NBUF, (int(per_pair * 0.1875) // _NBUF) * _NBUF)
    k0 = per_pair - k1
    pk = jnp.stack(
        [srcp.reshape(tot_chunks, _CHA),
         dstp.reshape(tot_chunks, _CHA),
         lax.bitcast_convert_type(ewp, jnp.int32).reshape(
             tot_chunks, _CHA)],
        axis=1)
    x_pad = jnp.concatenate(
        [x_src, x_tar, jnp.zeros((npa - n, _DIM), x_src.dtype)])

    degp = _make_deg_kernel(nd_chunks, npd)(
        dstp.reshape(_NW, nd_chunks, _CH),
        ewp.reshape(_NW, nd_chunks, _CH))

    blk = _CH
    nblk = npa // blk
    xw = pl.pallas_call(
        _k1a_body,
        grid=(nblk,),
        in_specs=[
            pl.BlockSpec((blk, _DIM), lambda i: (i, 0)),
            pl.BlockSpec((_DIM, _DIM), lambda i: (0, 0)),
        ],
        out_specs=pl.BlockSpec((blk, _DIM), lambda i: (i, 0)),
        out_shape=jax.ShapeDtypeStruct((npa, _DIM), jnp.float32),
    )(x_pad, W)

    yw, ywb = pl.pallas_call(
        _k1b_body,
        grid=(nblk,),
        in_specs=[
            pl.BlockSpec((blk, _DIM), lambda i: (i, 0)),
            pl.BlockSpec((_NC, blk), lambda i: (0, i)),
        ],
        out_specs=[pl.BlockSpec((blk, _DIM), lambda i: (i, 0)),
                   pl.BlockSpec((blk, _DIM), lambda i: (i, 0))],
        out_shape=[jax.ShapeDtypeStruct((npa, _DIM), jnp.float32),
                   jax.ShapeDtypeStruct((npa, _DIM), jnp.bfloat16)],
    )(xw, degp)

    acc = _make_agg_kernel(k0, k1, npa)(pk, ywb)

    xn = pl.pallas_call(
        _k3_body,
        grid=(nblk,),
        in_specs=[
            pl.BlockSpec((_NC, blk, _DIM), lambda i: (0, i, 0)),
            pl.BlockSpec((blk, _DIM), lambda i: (i, 0)),
            pl.BlockSpec((_NC, blk), lambda i: (0, i)),
            pl.BlockSpec((1, _DIM), lambda i: (0, 0)),
        ],
        out_specs=pl.BlockSpec((blk, _DIM), lambda i: (i, 0)),
        out_shape=jax.ShapeDtypeStruct((npa, _DIM), jnp.float32),
    )(acc, yw, degp, b.reshape(1, _DIM))

    return (xn[:n_src], xn[n_src:n])


# speedup vs baseline: 1.1000x; 1.1000x over previous
import functools

import jax
import jax.numpy as jnp
from jax import lax
from jax.experimental import pallas as pl
from jax.experimental.pallas import tpu as pltpu
from jax.experimental.pallas import tpu_sc as plsc

_NC = 2
_NS = 16
_L = 16
_NW = _NC * _NS
_CH = 128
_CHA = 64
_NBUF = 4
_DIM = 128
_EPS = 1e-06


def _mesh():
    return plsc.VectorSubcoreMesh(core_axis_name="c", subcore_axis_name="s")


def _sc_params():
    return pltpu.CompilerParams(needs_layout_passes=False)


def _make_deg_kernel(n_chunks, n_pad):
    trows = n_pad // _NS

    @functools.partial(
        pl.kernel,
        out_type=jax.ShapeDtypeStruct((_NC, n_pad), jnp.float32),
        mesh=_mesh(),
        compiler_params=_sc_params(),
        scratch_types=[
            pltpu.VMEM((n_chunks, _CH), jnp.int32),
            pltpu.VMEM((n_chunks, _CH), jnp.float32),
            pltpu.VMEM((trows,), jnp.float32),
            pltpu.VMEM_SHARED((n_pad,), jnp.float32),
            pltpu.SemaphoreType.DMA,
        ],
    )
    def deg_kernel(dst_hbm, ew_hbm, out_hbm, dst_v, ew_v, zbuf, acc_sh, sem):
        cid = lax.axis_index("c")
        sid = lax.axis_index("s")
        wid = sid * _NC + cid

        def zinit(i, carry):
            zbuf[pl.ds(i * _L, _L)] = jnp.zeros((_L,), jnp.float32)
            return carry

        lax.fori_loop(0, trows // _L, zinit, 0)
        pltpu.sync_copy(zbuf, acc_sh.at[pl.ds(sid * trows, trows)])
        pltpu.sync_copy(dst_hbm.at[wid], dst_v)
        pltpu.sync_copy(ew_hbm.at[wid], ew_v)
        plsc.subcore_barrier()

        def fire(c, carry):
            pltpu.async_copy(ew_v.at[c], acc_sh.at[dst_v.at[c]], sem,
                             add=True)
            return carry

        lax.fori_loop(0, n_chunks, fire, 0)

        def drain(c, carry):
            pltpu.make_async_copy(
                ew_v.at[c], acc_sh.at[dst_v.at[c]], sem).wait()
            return carry

        lax.fori_loop(0, n_chunks, drain, 0)
        plsc.subcore_barrier()
        pltpu.sync_copy(acc_sh.at[pl.ds(sid * trows, trows)],
                        out_hbm.at[cid, pl.ds(sid * trows, trows)])

    return deg_kernel


def _make_agg_kernel(k0, k1, n_pad):
    trows = n_pad // _NS
    zfull = trows // _CHA
    zrem = trows - zfull * _CHA

    @functools.partial(
        pl.kernel,
        out_type=jax.ShapeDtypeStruct((_NC, n_pad, _DIM), jnp.float32),
        mesh=_mesh(),
        compiler_params=_sc_params(),
        scratch_types=[pltpu.VMEM((_CHA, _DIM), jnp.float32)
                       for _ in range(_NBUF)]
        + [pltpu.VMEM((3, _CHA), jnp.int32) for _ in range(_NBUF)]
        + [pltpu.VMEM_SHARED((n_pad, _DIM), jnp.float32)]
        + [pltpu.SemaphoreType.DMA for _ in range(3 * _NBUF)],
    )
    def agg_kernel(pk_hbm, yw_hbm, out_hbm, *rest):
        rows = rest[:_NBUF]
        pk = rest[_NBUF:2 * _NBUF]
        acc_sh = rest[2 * _NBUF]
        sem_g = rest[2 * _NBUF + 1:3 * _NBUF + 1]
        sem_s = rest[3 * _NBUF + 1:4 * _NBUF + 1]
        sem_i = rest[4 * _NBUF + 1:]
        cid = lax.axis_index("c")
        sid = lax.axis_index("s")
        nck = jnp.where(cid == 0, k0, k1)
        base = jnp.where(cid == 0, sid * k0, _NS * k0 + sid * k1)
        n_outer = nck // _NBUF

        def zinit(r, carry):
            for k in range(_DIM // _L):
                rows[0][r, pl.ds(k * _L, _L)] = jnp.zeros((_L,), jnp.float32)
            return carry

        lax.fori_loop(0, _CHA, zinit, 0)
        for t in range(zfull):
            pltpu.sync_copy(rows[0],
                            acc_sh.at[pl.ds(sid * trows + t * _CHA, _CHA)])
        if zrem:
            pltpu.sync_copy(
                rows[0].at[pl.ds(0, zrem)],
                acc_sh.at[pl.ds(sid * trows + zfull * _CHA, zrem)])
        plsc.subcore_barrier()

        def pk_load(c, b):
            pltpu.async_copy(pk_hbm.at[base + c], pk[b], sem_i[b])

        def wait_pk(c, b):
            pltpu.make_async_copy(pk_hbm.at[base + c], pk[b],
                                  sem_i[b]).wait()

        def gather(c, b):
            pltpu.async_copy(yw_hbm.at[pk[b].at[0]], rows[b], sem_g[b])

        def wait_gather(c, b):
            pltpu.make_async_copy(
                yw_hbm.at[pk[b].at[0]], rows[b], sem_g[b]).wait()

        def scatter(c, b):
            pltpu.async_copy(rows[b], acc_sh.at[pk[b].at[1]], sem_s[b],
                             add=True)

        def wait_scatter(c, b):
            pltpu.make_async_copy(
                rows[b], acc_sh.at[pk[b].at[1]], sem_s[b]).wait()

        pk_load(0, 0)
        pk_load(1, 1)
        wait_pk(0, 0)
        gather(0, 0)

        def outer(g, carry):
            for b in range(_NBUF):
                c = g * _NBUF + b
                wait_gather(c, b)

                @pl.when(c >= 2)
                def _():
                    wait_scatter(c - 2, (b + 2) % _NBUF)

                @pl.when(c + 2 < nck)
                def _():
                    pk_load(c + 2, (b + 2) % _NBUF)

                @pl.when(c + 1 < nck)
                def _():
                    wait_pk(c + 1, (b + 1) % _NBUF)
                    gather(c + 1, (b + 1) % _NBUF)

                @plsc.parallel_loop(0, _CHA // _L, unroll=2)
                def grp(gr):
                    for j in range(_L):
                        r = gr * _L + j
                        sp = plsc.bitcast(
                            plsc.load_gather(
                                pk[b].at[2],
                                [jnp.full((_L,), r, jnp.int32)]),
                            jnp.float32)
                        for k in range(_DIM // _L):
                            rows[b][r, pl.ds(k * _L, _L)] = (
                                rows[b][r, pl.ds(k * _L, _L)] * sp)

                scatter(c, b)
            return carry

        lax.fori_loop(0, n_outer, outer, 0)
        wait_scatter(nck - 2, (_NBUF - 2) % _NBUF)
        wait_scatter(nck - 1, _NBUF - 1)
        plsc.subcore_barrier()
        pltpu.sync_copy(acc_sh.at[pl.ds(sid * trows, trows)],
                        out_hbm.at[cid, pl.ds(sid * trows, trows)])

    return agg_kernel


def _dis_from_parts(degp_block):
    deg = 1.0 + degp_block[0, :] + degp_block[1, :]
    safe = jnp.where(deg > 0, deg, 1.0)
    return jnp.where(deg > 0, lax.rsqrt(safe), 0.0)


def _k1a_body(x_ref, w_ref, xw_ref):
    xw_ref[...] = lax.dot_general(x_ref[...], w_ref[...],
                                  (((1,), (1,)), ((), ())),
                                  preferred_element_type=jnp.float32)


def _k1b_body(xw_ref, degp_ref, yw_ref):
    dis = _dis_from_parts(degp_ref[...])
    yw_ref[...] = xw_ref[...] * dis[:, None]


def _k3_body(acc_ref, yw_ref, degp_ref, b_ref, o_ref):
    dis = _dis_from_parts(degp_ref[...])
    out = (acc_ref[0] + acc_ref[1] + yw_ref[...]) * dis[:, None] + b_ref[...]
    mean = jnp.mean(out, axis=1, keepdims=True)
    cen = out - mean
    var = jnp.mean(cen * cen, axis=1, keepdims=True)
    xn = cen * lax.rsqrt(var + _EPS)
    o_ref[...] = jnp.where(xn >= 0, xn, 0.01 * xn)


def kernel(x_src, x_tar, edge_index, edge_weight, W, b):
    n_src = x_src.shape[0]
    n_tar = x_tar.shape[0]
    n = n_src + n_tar
    e = edge_index.shape[1]

    gran = _NW * _CHA * _NBUF
    e_pad = ((e + gran - 1) // gran) * gran
    nd_chunks = e_pad // (_NW * _CH)
    npd = ((n + 255) // 256) * 256
    npa = ((n + 127) // 128) * 128

    src = edge_index[0]
    dst = edge_index[1]
    ep = e_pad - e
    srcp = jnp.concatenate([src, jnp.zeros((ep,), src.dtype)])
    dstp = jnp.concatenate([dst, jnp.zeros((ep,), dst.dtype)])
    ewp = jnp.concatenate(
        [edge_weight, jnp.zeros((ep,), edge_weight.dtype)])
    tot_chunks = e_pad // _CHA
    per_pair = tot_chunks // _NS
    k1 = max(_NBUF, (int(per_pair * 0.1875) // _NBUF) * _NBUF)
    k0 = per_pair - k1
    pk = jnp.stack(
        [srcp.reshape(tot_chunks, _CHA),
         dstp.reshape(tot_chunks, _CHA),
         lax.bitcast_convert_type(ewp, jnp.int32).reshape(
             tot_chunks, _CHA)],
        axis=1)
    x_pad = jnp.concatenate(
        [x_src, x_tar, jnp.zeros((npa - n, _DIM), x_src.dtype)])

    degp = _make_deg_kernel(nd_chunks, npd)(
        dstp.reshape(_NW, nd_chunks, _CH),
        ewp.reshape(_NW, nd_chunks, _CH))

    blk = _CH
    nblk = npa // blk
    xw = pl.pallas_call(
        _k1a_body,
        grid=(nblk,),
        in_specs=[
            pl.BlockSpec((blk, _DIM), lambda i: (i, 0)),
            pl.BlockSpec((_DIM, _DIM), lambda i: (0, 0)),
        ],
        out_specs=pl.BlockSpec((blk, _DIM), lambda i: (i, 0)),
        out_shape=jax.ShapeDtypeStruct((npa, _DIM), jnp.float32),
    )(x_pad, W)

    yw = pl.pallas_call(
        _k1b_body,
        grid=(nblk,),
        in_specs=[
            pl.BlockSpec((blk, _DIM), lambda i: (i, 0)),
            pl.BlockSpec((_NC, blk), lambda i: (0, i)),
        ],
        out_specs=pl.BlockSpec((blk, _DIM), lambda i: (i, 0)),
        out_shape=jax.ShapeDtypeStruct((npa, _DIM), jnp.float32),
    )(xw, degp)

    acc = _make_agg_kernel(k0, k1, npa)(pk, yw)

    xn = pl.pallas_call(
        _k3_body,
        grid=(nblk,),
        in_specs=[
            pl.BlockSpec((_NC, blk, _DIM), lambda i: (0, i, 0)),
            pl.BlockSpec((blk, _DIM), lambda i: (i, 0)),
            pl.BlockSpec((_NC, blk), lambda i: (0, i)),
            pl.BlockSpec((1, _DIM), lambda i: (0, 0)),
        ],
        out_specs=pl.BlockSpec((blk, _DIM), lambda i: (i, 0)),
        out_shape=jax.ShapeDtypeStruct((npa, _DIM), jnp.float32),
    )(acc, yw, degp, b.reshape(1, _DIM))

    return (xn[:n_src], xn[n_src:n])
